# baseline (speedup 1.0000x reference)
import functools

import jax
import jax.numpy as jnp
from jax import lax
from jax.experimental import pallas as pl
from jax.experimental.pallas import tpu as pltpu
from jax.experimental.pallas import tpu_sc as plsc

_NC = 2
_NS = 16
_L = 16
_NW = _NC * _NS


@functools.lru_cache(maxsize=None)
def _build(P, H, W, C):
    PW = P // _NW
    n_chunks = PW // C
    mesh = plsc.VectorSubcoreMesh(
        core_axis_name="c", subcore_axis_name="s",
        num_cores=_NC, num_subcores=_NS)

    vmem_f = lambda n: pltpu.VMEM((n,), jnp.float32)
    vmem_i = lambda n: pltpu.VMEM((n,), jnp.int32)

    @functools.partial(
        pl.kernel,
        out_type=jax.ShapeDtypeStruct((P,), jnp.float32),
        mesh=mesh,
        scratch_types=[
            [[vmem_f(C), vmem_f(C)]] * 2,
            [vmem_i(4 * C)] * 2,
            [vmem_f(4 * C)] * 2,
            [vmem_f(C)] * 2,
            [vmem_f(C)] * 2,
            [vmem_f(C)] * 2,
            [pltpu.SemaphoreType.DMA] * 2,
            [pltpu.SemaphoreType.DMA] * 2,
            [pltpu.SemaphoreType.DMA] * 2,
        ],
    )
    def grid_sample(x_hbm, y_hbm, g_hbm, out_hbm, xyv, idxv, valv, wxv,
                    wyv, outv, sx, sg, so):
        wid = lax.axis_index("s") * _NC + lax.axis_index("c")
        base0 = wid * PW
        fw = jnp.float32(W - 1)
        fh = jnp.float32(H - 1)
        lane = lax.iota(jnp.int32, _L)

        def _vperm(v, idx):
            dn = lax.GatherDimensionNumbers(
                offset_dims=(), collapsed_slice_dims=(0,),
                start_index_map=(0,))
            return lax.gather(v, idx[:, None], dn, slice_sizes=(1,),
                              mode=lax.GatherScatterMode.PROMISE_IN_BOUNDS)

        def start_load(k, b):
            base = base0 + k * C
            a = pltpu.async_copy(x_hbm.at[pl.ds(base, C)], xyv[b][0],
                                 sx[b])
            c = pltpu.async_copy(y_hbm.at[pl.ds(base, C)], xyv[b][1],
                                 sx[b])
            return (a, c)

        def compute_idx(b):
            @pl.loop(0, C // _L)
            def _indices(j):
                o = j * _L
                xf = xyv[b][0][pl.ds(o, _L)] * fw
                yf = xyv[b][1][pl.ds(o, _L)] * fh
                x0 = jnp.clip(xf.astype(jnp.int32), 0, W - 1)
                y0 = jnp.clip(yf.astype(jnp.int32), 0, H - 1)
                wxv[b][pl.ds(o, _L)] = xf - x0.astype(jnp.float32)
                wyv[b][pl.ds(o, _L)] = yf - y0.astype(jnp.float32)
                x1 = jnp.minimum(x0 + 1, W - 1)
                r0 = y0 * W
                r1 = jnp.minimum(y0 + 1, H - 1) * W
                corner = [r0 + x0, r0 + x1, r1 + x0, r1 + x1]
                psel = lane >> 2
                csel = lane & 3
                for q in range(4):
                    g = [_vperm(cv, psel + 4 * q) for cv in corner]
                    quad = jnp.where(
                        csel == 0, g[0],
                        jnp.where(csel == 1, g[1],
                                  jnp.where(csel == 2, g[2], g[3])))
                    idxv[b][pl.ds(4 * o + q * _L, _L)] = quad

        def start_gather(b):
            return pltpu.async_copy(g_hbm.at[idxv[b]], valv[b], sg[b])

        def blend(b):
            @pl.loop(0, C // _L)
            def _blend(j):
                o = j * _L
                quads = [valv[b][pl.ds(4 * o + q * _L, _L)] for q in range(4)]
                csel = lane & 3
                v = []
                for c in range(4):
                    g = [_vperm(qv, csel * 4 + c) for qv in quads]
                    v.append(jnp.where(
                        lane < 4, g[0],
                        jnp.where(lane < 8, g[1],
                                  jnp.where(lane < 12, g[2], g[3]))))
                wx = wxv[b][pl.ds(o, _L)]
                wy = wyv[b][pl.ds(o, _L)]
                top = v[0] + wx * (v[1] - v[0])
                bot = v[2] + wx * (v[3] - v[2])
                outv[b][pl.ds(o, _L)] = top + wy * (bot - top)

        def start_store(k, b):
            base = base0 + k * C
            return pltpu.async_copy(outv[b], out_hbm.at[pl.ds(base, C)],
                                    so[b])

        loads = [None] * n_chunks
        gathers = [None] * n_chunks
        store_desc = [None, None]
        loads[0] = start_load(0, 0)
        for k in range(n_chunks):
            b = k % 2
            for d in loads[k]:
                d.wait()
            if k + 1 < n_chunks:
                loads[k + 1] = start_load(k + 1, 1 - b)
            compute_idx(b)
            if k >= 1:
                gathers[k - 1].wait()
            gathers[k] = start_gather(b)
            if k >= 1:
                if store_desc[1 - b] is not None:
                    store_desc[1 - b].wait()
                blend(1 - b)
                store_desc[1 - b] = start_store(k - 1, 1 - b)
        bl = (n_chunks - 1) % 2
        gathers[n_chunks - 1].wait()
        if store_desc[bl] is not None:
            store_desc[bl].wait()
        blend(bl)
        start_store(n_chunks - 1, bl).wait()
        if store_desc[1 - bl] is not None:
            store_desc[1 - bl].wait()

    return grid_sample


def kernel(xy, grid):
    P = xy.shape[0]
    H, W = grid.shape[-2], grid.shape[-1]
    return _build(P, H, W, 4096)(xy[:, 0], xy[:, 1], grid.reshape(-1))


# speedup vs baseline: 1.0282x; 1.0282x over previous
import functools

import jax
import jax.numpy as jnp
from jax import lax
from jax.experimental import pallas as pl
from jax.experimental.pallas import tpu as pltpu
from jax.experimental.pallas import tpu_sc as plsc

_NC = 2
_NS = 16
_L = 16
_NW = _NC * _NS


@functools.lru_cache(maxsize=None)
def _build(P, H, W, C):
    PW = P // _NW
    n_chunks = PW // C
    mesh = plsc.VectorSubcoreMesh(
        core_axis_name="c", subcore_axis_name="s",
        num_cores=_NC, num_subcores=_NS)

    vmem_f = lambda n: pltpu.VMEM((n,), jnp.float32)
    vmem_i = lambda n: pltpu.VMEM((n,), jnp.int32)

    @functools.partial(
        pl.kernel,
        out_type=jax.ShapeDtypeStruct((P,), jnp.float32),
        mesh=mesh,
        scratch_types=[
            [[vmem_f(C), vmem_f(C)]] * 2,
            [vmem_i(4 * C)] * 2,
            [vmem_f(4 * C)] * 2,
            [vmem_f(C)] * 2,
            [vmem_f(C)] * 2,
            [vmem_f(C)] * 2,
            [pltpu.SemaphoreType.DMA] * 2,
            [pltpu.SemaphoreType.DMA] * 2,
            [pltpu.SemaphoreType.DMA] * 2,
        ],
    )
    def grid_sample(xy_hbm, g_hbm, out_hbm, xyv, idxv, valv, wxv, wyv,
                    outv, sx, sg, so):
        wid = lax.axis_index("s") * _NC + lax.axis_index("c")
        base0 = wid * PW
        fw = jnp.float32(W - 1)
        fh = jnp.float32(H - 1)
        lane = lax.iota(jnp.int32, _L)

        def _vperm(v, idx):
            dn = lax.GatherDimensionNumbers(
                offset_dims=(), collapsed_slice_dims=(0,),
                start_index_map=(0,))
            return lax.gather(v, idx[:, None], dn, slice_sizes=(1,),
                              mode=lax.GatherScatterMode.PROMISE_IN_BOUNDS)

        def start_load(k, b):
            base = base0 + k * C
            a = pltpu.async_copy(xy_hbm.at[0, pl.ds(base, C)], xyv[b][0],
                                 sx[b])
            c = pltpu.async_copy(xy_hbm.at[1, pl.ds(base, C)], xyv[b][1],
                                 sx[b])
            return (a, c)

        def compute_idx(b):
            @pl.loop(0, C // _L)
            def _indices(j):
                o = j * _L
                xf = xyv[b][0][pl.ds(o, _L)] * fw
                yf = xyv[b][1][pl.ds(o, _L)] * fh
                x0 = jnp.clip(xf.astype(jnp.int32), 0, W - 1)
                y0 = jnp.clip(yf.astype(jnp.int32), 0, H - 1)
                wxv[b][pl.ds(o, _L)] = xf - x0.astype(jnp.float32)
                wyv[b][pl.ds(o, _L)] = yf - y0.astype(jnp.float32)
                x1 = jnp.minimum(x0 + 1, W - 1)
                r0 = y0 * W
                r1 = jnp.minimum(y0 + 1, H - 1) * W
                corner = [r0 + x0, r0 + x1, r1 + x0, r1 + x1]
                psel = lane >> 2
                csel = lane & 3
                for q in range(4):
                    g = [_vperm(cv, psel + 4 * q) for cv in corner]
                    quad = jnp.where(
                        csel == 0, g[0],
                        jnp.where(csel == 1, g[1],
                                  jnp.where(csel == 2, g[2], g[3])))
                    idxv[b][pl.ds(4 * o + q * _L, _L)] = quad

        def start_gather(b):
            return pltpu.async_copy(g_hbm.at[idxv[b]], valv[b], sg[b])

        def blend(b):
            @pl.loop(0, C // _L)
            def _blend(j):
                o = j * _L
                quads = [valv[b][pl.ds(4 * o + q * _L, _L)] for q in range(4)]
                csel = lane & 3
                v = []
                for c in range(4):
                    g = [_vperm(qv, csel * 4 + c) for qv in quads]
                    v.append(jnp.where(
                        lane < 4, g[0],
                        jnp.where(lane < 8, g[1],
                                  jnp.where(lane < 12, g[2], g[3]))))
                wx = wxv[b][pl.ds(o, _L)]
                wy = wyv[b][pl.ds(o, _L)]
                top = v[0] + wx * (v[1] - v[0])
                bot = v[2] + wx * (v[3] - v[2])
                outv[b][pl.ds(o, _L)] = top + wy * (bot - top)

        def start_store(k, b):
            base = base0 + k * C
            return pltpu.async_copy(outv[b], out_hbm.at[pl.ds(base, C)],
                                    so[b])

        loads = [None] * n_chunks
        gathers = [None] * n_chunks
        store_desc = [None, None]
        loads[0] = start_load(0, 0)
        for k in range(n_chunks):
            b = k % 2
            for d in loads[k]:
                d.wait()
            if k + 1 < n_chunks:
                loads[k + 1] = start_load(k + 1, 1 - b)
            compute_idx(b)
            if k >= 1:
                gathers[k - 1].wait()
            gathers[k] = start_gather(b)
            if k >= 1:
                if store_desc[1 - b] is not None:
                    store_desc[1 - b].wait()
                blend(1 - b)
                store_desc[1 - b] = start_store(k - 1, 1 - b)
        bl = (n_chunks - 1) % 2
        gathers[n_chunks - 1].wait()
        if store_desc[bl] is not None:
            store_desc[bl].wait()
        blend(bl)
        start_store(n_chunks - 1, bl).wait()
        if store_desc[1 - bl] is not None:
            store_desc[1 - bl].wait()

    return grid_sample


def kernel(xy, grid):
    P = xy.shape[0]
    H, W = grid.shape[-2], grid.shape[-1]
    return _build(P, H, W, 4096)(xy.T, grid.reshape(-1))
